# baseline (speedup 1.0000x reference)
import jax
import jax.numpy as jnp
from jax import lax
from jax.experimental import pallas as pl
from jax.experimental.pallas import tpu as pltpu
from jax.experimental.pallas import tpu_sc as plsc

EMBED_DIM = 300
ROW_W = 384
NUM_CORES = 2
NUM_SUBCORES = 16
NUM_WORKERS = NUM_CORES * NUM_SUBCORES
CHUNK = 128
BLK = 128


def _transpose_body(mt_hbm, r_hbm, mbuf, rbuf, sem):
    D, V = mt_hbm.shape
    nblocks = V // BLK
    per_w = (nblocks + NUM_WORKERS - 1) // NUM_WORKERS
    wid = lax.axis_index("s") * NUM_CORES + lax.axis_index("c")
    iota = lax.iota(jnp.int32, 16)
    diag = [(iota + d) & 15 for d in range(16)]

    def block_step(k, carry):
        v = k * NUM_WORKERS + wid

        @pl.when(v < nblocks)
        def _():
            pltpu.sync_copy(mt_hbm.at[:, pl.ds(v * BLK, BLK)], mbuf)

            def cgroup(ci, c1):
                c0 = ci * 16
                cvec = c0 + iota
                cmask = cvec < D

                def lgroup(li, c2):
                    l0 = li * 16
                    for d in range(16):
                        lvec = l0 + diag[d]
                        x = plsc.load_gather(mbuf, [cvec, lvec], mask=cmask)
                        plsc.store_scatter(rbuf, [lvec, cvec], x, mask=cmask)
                    return c2

                lax.fori_loop(0, BLK // 16, lgroup, 0)
                return c1

            lax.fori_loop(0, (D + 15) // 16, cgroup, 0)
            pltpu.sync_copy(rbuf, r_hbm.at[pl.ds(v * BLK, BLK)])

        return carry

    lax.fori_loop(0, per_w, block_step, 0)


def _gather_body(ids_hbm, table_hbm, out_hbm, idx_v, rows_v, sem):
    chunks_per_w = ids_hbm.shape[1]
    wid = lax.axis_index("s") * NUM_CORES + lax.axis_index("c")
    base = wid * chunks_per_w
    pltpu.sync_copy(ids_hbm.at[wid], idx_v)

    def step(j, carry):
        idx_row = idx_v.at[j]
        cps = [
            pltpu.async_copy(
                table_hbm.at[idx_row, pl.ds(c * 128, 128)],
                rows_v.at[:, pl.ds(c * 128, 128)], sem)
            for c in range(ROW_W // 128)
        ]
        for cp in cps:
            cp.wait()
        pltpu.sync_copy(rows_v, out_hbm.at[pl.ds((base + j) * CHUNK, CHUNK)])
        return carry

    lax.fori_loop(0, chunks_per_w, step, 0)


def kernel(token_ids, glove_table):
    B, S = token_ids.shape
    V, D = glove_table.shape
    n = B * S
    assert n % (NUM_WORKERS * CHUNK) == 0
    n_chunks = n // CHUNK
    chunks_per_w = n_chunks // NUM_WORKERS
    ids = token_ids.astype(jnp.int32).reshape(NUM_WORKERS, chunks_per_w, CHUNK)

    run_t = pl.kernel(
        _transpose_body,
        out_type=jax.ShapeDtypeStruct((V, ROW_W), jnp.float32),
        mesh=plsc.VectorSubcoreMesh(core_axis_name="c", subcore_axis_name="s"),
        scratch_types=[
            pltpu.VMEM((D, BLK), jnp.float32),
            pltpu.VMEM((BLK, ROW_W), jnp.float32),
            pltpu.SemaphoreType.DMA,
        ],
        compiler_params=pltpu.CompilerParams(needs_layout_passes=False),
    )
    table_rm = run_t(glove_table.T)

    run_g = pl.kernel(
        _gather_body,
        out_type=jax.ShapeDtypeStruct((n, ROW_W), jnp.float32),
        mesh=plsc.VectorSubcoreMesh(core_axis_name="c", subcore_axis_name="s"),
        scratch_types=[
            pltpu.VMEM((chunks_per_w, CHUNK), jnp.int32),
            pltpu.VMEM((CHUNK, ROW_W), jnp.float32),
            pltpu.SemaphoreType.DMA,
        ],
    )
    wide = run_g(ids, table_rm)
    return lax.slice(wide, (0, 0), (n, EMBED_DIM)).reshape(B, S, EMBED_DIM)


# speedup vs baseline: 1.1314x; 1.1314x over previous
import jax
import jax.numpy as jnp
from jax import lax
from jax.experimental import pallas as pl
from jax.experimental.pallas import tpu as pltpu
from jax.experimental.pallas import tpu_sc as plsc

EMBED_DIM = 300
ROW_W = 384
NUM_CORES = 2
NUM_SUBCORES = 16
NUM_WORKERS = NUM_CORES * NUM_SUBCORES
CHUNK = 128
BLK = 128


def _transpose_body(mt_hbm, r_hbm, mb0, mb1, rbuf, sem):
    D, V = mt_hbm.shape
    nblocks = V // BLK
    per_w = (nblocks + NUM_WORKERS - 1) // NUM_WORKERS
    assert per_w % 2 == 0
    wid = lax.axis_index("s") * NUM_CORES + lax.axis_index("c")
    iota = lax.iota(jnp.int32, 16)
    diag = [(iota + d) & 15 for d in range(16)]
    full_groups = D // 16
    tail_c0 = full_groups * 16
    tail_mask = (tail_c0 + iota) < D

    def transpose_block(mbuf):
        def cgroup(ci, c1):
            cvec = ci * 16 + iota

            def lgroup(li, c2):
                l0 = li * 16
                for d in range(16):
                    lvec = l0 + diag[d]
                    x = plsc.load_gather(mbuf, [cvec, lvec])
                    plsc.store_scatter(rbuf, [lvec, cvec], x)
                return c2

            lax.fori_loop(0, BLK // 16, lgroup, 0)
            return c1

        lax.fori_loop(0, full_groups, cgroup, 0)
        cvec = tail_c0 + iota

        def ltail(li, c2):
            l0 = li * 16
            for d in range(16):
                lvec = l0 + diag[d]
                x = plsc.load_gather(mbuf, [cvec, lvec], mask=tail_mask)
                plsc.store_scatter(rbuf, [lvec, cvec], x, mask=tail_mask)
            return c2

        lax.fori_loop(0, BLK // 16, ltail, 0)

    def start_in(v, mbuf):
        @pl.when(v < nblocks)
        def _():
            pltpu.async_copy(mt_hbm.at[:, pl.ds(v * BLK, BLK)], mbuf, sem)

    def wait_in(v, mbuf):
        @pl.when(v < nblocks)
        def _():
            pltpu.make_async_copy(mt_hbm.at[:, pl.ds(v * BLK, BLK)], mbuf,
                                  sem).wait()

    def do_block(v, mbuf):
        @pl.when(v < nblocks)
        def _():
            transpose_block(mbuf)
            pltpu.sync_copy(rbuf, r_hbm.at[pl.ds(v * BLK, BLK)])

    start_in(wid, mb0)

    def pair_step(g, carry):
        v0 = (2 * g) * NUM_WORKERS + wid
        v1 = v0 + NUM_WORKERS
        v2 = v1 + NUM_WORKERS
        wait_in(v0, mb0)
        start_in(v1, mb1)
        do_block(v0, mb0)
        wait_in(v1, mb1)
        start_in(v2, mb0)
        do_block(v1, mb1)
        return carry

    lax.fori_loop(0, per_w // 2, pair_step, 0)


def _gather_body(ids_hbm, table_hbm, out_hbm, idx_v, rows_v, sem):
    chunks_per_w = ids_hbm.shape[1]
    wid = lax.axis_index("s") * NUM_CORES + lax.axis_index("c")
    base = wid * chunks_per_w
    pltpu.sync_copy(ids_hbm.at[wid], idx_v)

    def step(j, carry):
        idx_row = idx_v.at[j]
        cps = [
            pltpu.async_copy(
                table_hbm.at[idx_row, pl.ds(c * 128, 128)],
                rows_v.at[:, pl.ds(c * 128, 128)], sem)
            for c in range(ROW_W // 128)
        ]
        for cp in cps:
            cp.wait()
        pltpu.sync_copy(rows_v, out_hbm.at[pl.ds((base + j) * CHUNK, CHUNK)])
        return carry

    lax.fori_loop(0, chunks_per_w, step, 0)


def kernel(token_ids, glove_table):
    B, S = token_ids.shape
    V, D = glove_table.shape
    n = B * S
    assert n % (NUM_WORKERS * CHUNK) == 0
    n_chunks = n // CHUNK
    chunks_per_w = n_chunks // NUM_WORKERS
    ids = token_ids.astype(jnp.int32).reshape(NUM_WORKERS, chunks_per_w, CHUNK)

    run_t = pl.kernel(
        _transpose_body,
        out_type=jax.ShapeDtypeStruct((V, ROW_W), jnp.float32),
        mesh=plsc.VectorSubcoreMesh(core_axis_name="c", subcore_axis_name="s"),
        scratch_types=[
            pltpu.VMEM((D, BLK), jnp.float32),
            pltpu.VMEM((D, BLK), jnp.float32),
            pltpu.VMEM((BLK, ROW_W), jnp.float32),
            pltpu.SemaphoreType.DMA,
        ],
        compiler_params=pltpu.CompilerParams(
            needs_layout_passes=False, disable_bounds_checks=True),
    )
    table_rm = run_t(glove_table.T)

    run_g = pl.kernel(
        _gather_body,
        out_type=jax.ShapeDtypeStruct((n, ROW_W), jnp.float32),
        mesh=plsc.VectorSubcoreMesh(core_axis_name="c", subcore_axis_name="s"),
        scratch_types=[
            pltpu.VMEM((chunks_per_w, CHUNK), jnp.int32),
            pltpu.VMEM((CHUNK, ROW_W), jnp.float32),
            pltpu.SemaphoreType.DMA,
        ],
    )
    wide = run_g(ids, table_rm)
    return lax.slice(wide, (0, 0), (n, EMBED_DIM)).reshape(B, S, EMBED_DIM)


# speedup vs baseline: 1.4529x; 1.2842x over previous
import jax
import jax.numpy as jnp
from jax import lax
from jax.experimental import pallas as pl
from jax.experimental.pallas import tpu as pltpu
from jax.experimental.pallas import tpu_sc as plsc

EMBED_DIM = 300
ROW_W = 384
NUM_CORES = 2
NUM_SUBCORES = 16
NUM_WORKERS = NUM_CORES * NUM_SUBCORES
CHUNK = 128
BLK = 128


def _transpose_body(mt_hbm, r_hbm, mb0, mb1, rbuf, sem):
    D, V = mt_hbm.shape
    nblocks = V // BLK
    per_w = (nblocks + NUM_WORKERS - 1) // NUM_WORKERS
    assert per_w % 2 == 0
    wid = lax.axis_index("s") * NUM_CORES + lax.axis_index("c")
    iota = lax.iota(jnp.int32, 16)
    full_groups = D // 16
    tail_c0 = full_groups * 16
    stages = (1, 2, 4, 8)
    rot_fwd = {s: (iota - s) & 15 for s in stages}
    rot_bwd = {s: (iota + s) & 15 for s in stages}
    lane_lo = {s: (iota & s) == 0 for s in stages}

    dnums = lax.GatherDimensionNumbers(
        offset_dims=(), collapsed_slice_dims=(0,), start_index_map=(0,))

    def vtake(x, idx):
        return lax.gather(x, idx[:, None], dnums, (1,),
                          mode=lax.GatherScatterMode.PROMISE_IN_BOUNDS)

    def bf16x16(v):
        for s in stages:
            nv = list(v)
            for k in range(16):
                if k & s:
                    continue
                a, b = v[k], v[k | s]
                rb = vtake(b, rot_fwd[s])
                ra = vtake(a, rot_bwd[s])
                nv[k] = jnp.where(lane_lo[s], a, rb)
                nv[k | s] = jnp.where(lane_lo[s], ra, b)
            v = nv
        return v

    def do_group(mbuf, rows, c0):
        def lgroup(li, c2):
            l0 = pl.multiple_of(li * 16, 16)
            v = [mbuf[rows[k], pl.ds(l0, 16)] for k in range(16)]
            v = bf16x16(v)
            for j in range(16):
                rbuf[l0 + j, pl.ds(c0, 16)] = v[j]
            return c2

        lax.fori_loop(0, BLK // 16, lgroup, 0)

    def transpose_block(mbuf):
        def cgroup(ci, c1):
            c0 = pl.multiple_of(ci * 16, 16)
            do_group(mbuf, [c0 + k for k in range(16)], c0)
            return c1

        lax.fori_loop(0, full_groups, cgroup, 0)
        do_group(mbuf, [min(tail_c0 + k, D - 1) for k in range(16)], tail_c0)

    def start_in(v, mbuf):
        @pl.when(v < nblocks)
        def _():
            pltpu.async_copy(mt_hbm.at[:, pl.ds(v * BLK, BLK)], mbuf, sem)

    def wait_in(v, mbuf):
        @pl.when(v < nblocks)
        def _():
            pltpu.make_async_copy(mt_hbm.at[:, pl.ds(v * BLK, BLK)], mbuf,
                                  sem).wait()

    def do_block(v, mbuf):
        @pl.when(v < nblocks)
        def _():
            transpose_block(mbuf)
            pltpu.sync_copy(rbuf, r_hbm.at[pl.ds(v * BLK, BLK)])

    start_in(wid, mb0)

    def pair_step(g, carry):
        v0 = (2 * g) * NUM_WORKERS + wid
        v1 = v0 + NUM_WORKERS
        v2 = v1 + NUM_WORKERS
        wait_in(v0, mb0)
        start_in(v1, mb1)
        do_block(v0, mb0)
        wait_in(v1, mb1)
        start_in(v2, mb0)
        do_block(v1, mb1)
        return carry

    lax.fori_loop(0, per_w // 2, pair_step, 0)


def _gather_body(ids_hbm, table_hbm, out_hbm, idx_v, rows_v, sem):
    chunks_per_w = ids_hbm.shape[1]
    wid = lax.axis_index("s") * NUM_CORES + lax.axis_index("c")
    base = wid * chunks_per_w
    pltpu.sync_copy(ids_hbm.at[wid], idx_v)

    def step(j, carry):
        idx_row = idx_v.at[j]
        cps = [
            pltpu.async_copy(
                table_hbm.at[idx_row, pl.ds(c * 128, 128)],
                rows_v.at[:, pl.ds(c * 128, 128)], sem)
            for c in range(ROW_W // 128)
        ]
        for cp in cps:
            cp.wait()
        pltpu.sync_copy(rows_v, out_hbm.at[pl.ds((base + j) * CHUNK, CHUNK)])
        return carry

    lax.fori_loop(0, chunks_per_w, step, 0)


def kernel(token_ids, glove_table):
    B, S = token_ids.shape
    V, D = glove_table.shape
    n = B * S
    assert n % (NUM_WORKERS * CHUNK) == 0
    n_chunks = n // CHUNK
    chunks_per_w = n_chunks // NUM_WORKERS
    ids = token_ids.astype(jnp.int32).reshape(NUM_WORKERS, chunks_per_w, CHUNK)

    run_t = pl.kernel(
        _transpose_body,
        out_type=jax.ShapeDtypeStruct((V, ROW_W), jnp.float32),
        mesh=plsc.VectorSubcoreMesh(core_axis_name="c", subcore_axis_name="s"),
        scratch_types=[
            pltpu.VMEM((D, BLK), jnp.float32),
            pltpu.VMEM((D, BLK), jnp.float32),
            pltpu.VMEM((BLK, ROW_W), jnp.float32),
            pltpu.SemaphoreType.DMA,
        ],
        compiler_params=pltpu.CompilerParams(
            needs_layout_passes=False, disable_bounds_checks=True),
    )
    table_rm = run_t(glove_table.T)

    run_g = pl.kernel(
        _gather_body,
        out_type=jax.ShapeDtypeStruct((n, ROW_W), jnp.float32),
        mesh=plsc.VectorSubcoreMesh(core_axis_name="c", subcore_axis_name="s"),
        scratch_types=[
            pltpu.VMEM((chunks_per_w, CHUNK), jnp.int32),
            pltpu.VMEM((CHUNK, ROW_W), jnp.float32),
            pltpu.SemaphoreType.DMA,
        ],
    )
    wide = run_g(ids, table_rm)
    return lax.slice(wide, (0, 0), (n, EMBED_DIM)).reshape(B, S, EMBED_DIM)


# speedup vs baseline: 1.6196x; 1.1148x over previous
import jax
import jax.numpy as jnp
from jax import lax
from jax.experimental import pallas as pl
from jax.experimental.pallas import tpu as pltpu
from jax.experimental.pallas import tpu_sc as plsc

EMBED_DIM = 300
CH_SPLIT = 128
R2_W = 256
ROW_W = 384
NUM_CORES = 2
NUM_SUBCORES = 16
NUM_WORKERS = NUM_CORES * NUM_SUBCORES
CHUNK = 128
BLK = 128


def _transpose_body(mt_hbm, r_hbm, mb0, mb1, rbuf, sem):
    D, V = mt_hbm.shape
    nblocks = V // BLK
    per_w = (nblocks + NUM_WORKERS - 1) // NUM_WORKERS
    assert per_w % 2 == 0
    wid = lax.axis_index("s") * NUM_CORES + lax.axis_index("c")
    iota = lax.iota(jnp.int32, 16)
    full_groups = D // 16
    tail_c0 = full_groups * 16
    stages = (1, 2, 4, 8)
    rot_fwd = {s: (iota - s) & 15 for s in stages}
    rot_bwd = {s: (iota + s) & 15 for s in stages}
    lane_lo = {s: (iota & s) == 0 for s in stages}
    dnums = lax.GatherDimensionNumbers(
        offset_dims=(), collapsed_slice_dims=(0,), start_index_map=(0,))

    def vtake(x, idx):
        return lax.gather(x, idx[:, None], dnums, (1,),
                          mode=lax.GatherScatterMode.PROMISE_IN_BOUNDS)

    def bf16x16(v):
        for s in stages:
            nv = list(v)
            for k in range(16):
                if k & s:
                    continue
                a, b = v[k], v[k | s]
                rb = vtake(b, rot_fwd[s])
                ra = vtake(a, rot_bwd[s])
                nv[k] = jnp.where(lane_lo[s], a, rb)
                nv[k | s] = jnp.where(lane_lo[s], ra, b)
            v = nv
        return v

    def do_group(mbuf, rows, cdst):
        def lgroup(li, c2):
            l0 = pl.multiple_of(li * 16, 16)
            v = [mbuf[rows[k], pl.ds(l0, 16)] for k in range(16)]
            v = bf16x16(v)
            for j in range(16):
                rbuf[l0 + j, pl.ds(cdst, 16)] = v[j]
            return c2

        lax.fori_loop(0, BLK // 16, lgroup, 0)

    def transpose_block(mbuf):
        def cgroup(ci, c1):
            c0 = pl.multiple_of(ci * 16, 16)
            do_group(mbuf, [c0 + k for k in range(16)],
                     pl.multiple_of(c0 - CH_SPLIT, 16))
            return c1

        lax.fori_loop(CH_SPLIT // 16, full_groups, cgroup, 0)
        do_group(mbuf, [min(tail_c0 + k, D - 1) for k in range(16)],
                 tail_c0 - CH_SPLIT)

    def start_in(v, mbuf):
        @pl.when(v < nblocks)
        def _():
            pltpu.async_copy(mt_hbm.at[:, pl.ds(v * BLK, BLK)], mbuf, sem)

    def wait_in(v, mbuf):
        @pl.when(v < nblocks)
        def _():
            pltpu.make_async_copy(mt_hbm.at[:, pl.ds(v * BLK, BLK)], mbuf,
                                  sem).wait()

    def do_block(v, mbuf):
        @pl.when(v < nblocks)
        def _():
            transpose_block(mbuf)
            pltpu.sync_copy(rbuf, r_hbm.at[pl.ds(v * BLK, BLK)])

    start_in(wid, mb0)

    def pair_step(g, carry):
        v0 = (2 * g) * NUM_WORKERS + wid
        v1 = v0 + NUM_WORKERS
        v2 = v1 + NUM_WORKERS
        wait_in(v0, mb0)
        start_in(v1, mb1)
        do_block(v0, mb0)
        wait_in(v1, mb1)
        start_in(v2, mb0)
        do_block(v1, mb1)
        return carry

    lax.fori_loop(0, per_w // 2, pair_step, 0)


def _gather_body(ids_hbm, r1_hbm, r2_hbm, out_hbm, idx_v, rows_v, sem):
    chunks_per_w = ids_hbm.shape[1]
    wid = lax.axis_index("s") * NUM_CORES + lax.axis_index("c")
    base = wid * chunks_per_w
    pltpu.sync_copy(ids_hbm.at[wid], idx_v)

    def step(j, carry):
        idx_row = idx_v.at[j]
        cps = [
            pltpu.async_copy(r1_hbm.at[idx_row],
                             rows_v.at[:, pl.ds(0, 128)], sem),
            pltpu.async_copy(r2_hbm.at[idx_row, pl.ds(0, 128)],
                             rows_v.at[:, pl.ds(128, 128)], sem),
            pltpu.async_copy(r2_hbm.at[idx_row, pl.ds(128, 128)],
                             rows_v.at[:, pl.ds(256, 128)], sem),
        ]
        for cp in cps:
            cp.wait()
        pltpu.sync_copy(rows_v, out_hbm.at[pl.ds((base + j) * CHUNK, CHUNK)])
        return carry

    lax.fori_loop(0, chunks_per_w, step, 0)


def kernel(token_ids, glove_table):
    B, S = token_ids.shape
    V, D = glove_table.shape
    n = B * S
    assert n % (NUM_WORKERS * CHUNK) == 0
    n_chunks = n // CHUNK
    chunks_per_w = n_chunks // NUM_WORKERS
    ids = token_ids.astype(jnp.int32).reshape(NUM_WORKERS, chunks_per_w, CHUNK)

    r1 = lax.slice(glove_table, (0, 0), (V, CH_SPLIT))

    run_t = pl.kernel(
        _transpose_body,
        out_type=jax.ShapeDtypeStruct((V, R2_W), jnp.float32),
        mesh=plsc.VectorSubcoreMesh(core_axis_name="c", subcore_axis_name="s"),
        scratch_types=[
            pltpu.VMEM((D, BLK), jnp.float32),
            pltpu.VMEM((D, BLK), jnp.float32),
            pltpu.VMEM((BLK, R2_W), jnp.float32),
            pltpu.SemaphoreType.DMA,
        ],
        compiler_params=pltpu.CompilerParams(
            needs_layout_passes=False, disable_bounds_checks=True),
    )
    r2 = run_t(glove_table.T)

    run_g = pl.kernel(
        _gather_body,
        out_type=jax.ShapeDtypeStruct((n, ROW_W), jnp.float32),
        mesh=plsc.VectorSubcoreMesh(core_axis_name="c", subcore_axis_name="s"),
        scratch_types=[
            pltpu.VMEM((chunks_per_w, CHUNK), jnp.int32),
            pltpu.VMEM((CHUNK, ROW_W), jnp.float32),
            pltpu.SemaphoreType.DMA,
        ],
    )
    wide = run_g(ids, r1, r2)
    return lax.slice(wide, (0, 0), (n, EMBED_DIM)).reshape(B, S, EMBED_DIM)


# speedup vs baseline: 1.6583x; 1.0239x over previous
import jax
import jax.numpy as jnp
from jax import lax
from jax.experimental import pallas as pl
from jax.experimental.pallas import tpu as pltpu
from jax.experimental.pallas import tpu_sc as plsc

EMBED_DIM = 300
CH_SPLIT = 128
R2_W = 256
ROW_W = 384
NUM_CORES = 2
NUM_SUBCORES = 16
NUM_WORKERS = NUM_CORES * NUM_SUBCORES
CHUNK = 128
BLK = 128


def _transpose_body(mt_hbm, r_hbm, mb0, mb1, rbuf, sem):
    D, V = mt_hbm.shape
    nblocks = V // BLK
    per_w = (nblocks + NUM_WORKERS - 1) // NUM_WORKERS
    assert per_w % 2 == 0
    wid = lax.axis_index("s") * NUM_CORES + lax.axis_index("c")
    iota = lax.iota(jnp.int32, 16)
    full_groups = D // 16
    tail_c0 = full_groups * 16
    stages = (1, 2, 4, 8)
    rot_fwd = {s: (iota - s) & 15 for s in stages}
    rot_bwd = {s: (iota + s) & 15 for s in stages}
    lane_lo = {s: (iota & s) == 0 for s in stages}
    dnums = lax.GatherDimensionNumbers(
        offset_dims=(), collapsed_slice_dims=(0,), start_index_map=(0,))

    def vtake(x, idx):
        return lax.gather(x, idx[:, None], dnums, (1,),
                          mode=lax.GatherScatterMode.PROMISE_IN_BOUNDS)

    def bf16x16(v):
        for s in stages:
            nv = list(v)
            for k in range(16):
                if k & s:
                    continue
                a, b = v[k], v[k | s]
                rb = vtake(b, rot_fwd[s])
                ra = vtake(a, rot_bwd[s])
                nv[k] = jnp.where(lane_lo[s], a, rb)
                nv[k | s] = jnp.where(lane_lo[s], ra, b)
            v = nv
        return v

    def do_group(mbuf, rows, cdst):
        def lgroup(li, c2):
            l0 = pl.multiple_of(li * 16, 16)
            v = [mbuf[rows[k], pl.ds(l0, 16)] for k in range(16)]
            v = bf16x16(v)
            for j in range(16):
                rbuf[l0 + j, pl.ds(cdst, 16)] = v[j]
            return c2

        lax.fori_loop(0, BLK // 16, lgroup, 0)

    def transpose_block(mbuf):
        def cgroup(ci, c1):
            c0 = pl.multiple_of(ci * 16, 16)
            do_group(mbuf, [c0 + k for k in range(16)],
                     pl.multiple_of(c0 - CH_SPLIT, 16))
            return c1

        lax.fori_loop(CH_SPLIT // 16, full_groups, cgroup, 0)
        do_group(mbuf, [min(tail_c0 + k, D - 1) for k in range(16)],
                 tail_c0 - CH_SPLIT)

    def start_in(v, mbuf):
        @pl.when(v < nblocks)
        def _():
            pltpu.async_copy(mt_hbm.at[:, pl.ds(v * BLK, BLK)], mbuf, sem)

    def wait_in(v, mbuf):
        @pl.when(v < nblocks)
        def _():
            pltpu.make_async_copy(mt_hbm.at[:, pl.ds(v * BLK, BLK)], mbuf,
                                  sem).wait()

    def do_block(v, mbuf):
        @pl.when(v < nblocks)
        def _():
            transpose_block(mbuf)
            pltpu.sync_copy(rbuf, r_hbm.at[pl.ds(v * BLK, BLK)])

    start_in(wid, mb0)

    def pair_step(g, carry):
        v0 = (2 * g) * NUM_WORKERS + wid
        v1 = v0 + NUM_WORKERS
        v2 = v1 + NUM_WORKERS
        wait_in(v0, mb0)
        start_in(v1, mb1)
        do_block(v0, mb0)
        wait_in(v1, mb1)
        start_in(v2, mb0)
        do_block(v1, mb1)
        return carry

    lax.fori_loop(0, per_w // 2, pair_step, 0)


def _gather_body(ids_hbm, r1_hbm, r2_hbm, out_hbm, idx_v, rv0, rv1, sem):
    chunks_per_w = ids_hbm.shape[1]
    assert chunks_per_w % 2 == 0
    wid = lax.axis_index("s") * NUM_CORES + lax.axis_index("c")
    base = wid * chunks_per_w
    pltpu.sync_copy(ids_hbm.at[wid], idx_v)

    def descriptors(j, rows_v):
        idx_row = idx_v.at[j]
        return [
            pltpu.make_async_copy(r1_hbm.at[idx_row],
                                  rows_v.at[:, pl.ds(0, 128)], sem),
            pltpu.make_async_copy(r2_hbm.at[idx_row, pl.ds(0, 128)],
                                  rows_v.at[:, pl.ds(128, 128)], sem),
            pltpu.make_async_copy(r2_hbm.at[idx_row, pl.ds(128, 128)],
                                  rows_v.at[:, pl.ds(256, 128)], sem),
        ]

    def start(j, rows_v):
        @pl.when(j < chunks_per_w)
        def _():
            for cp in descriptors(j, rows_v):
                cp.start()

    def finish(j, rows_v):
        for cp in descriptors(j, rows_v):
            cp.wait()
        pltpu.sync_copy(rows_v, out_hbm.at[pl.ds((base + j) * CHUNK, CHUNK)])

    start(0, rv0)

    def pair_step(g, carry):
        j0 = 2 * g
        start(j0 + 1, rv1)
        finish(j0, rv0)
        start(j0 + 2, rv0)
        finish(j0 + 1, rv1)
        return carry

    lax.fori_loop(0, chunks_per_w // 2, pair_step, 0)


def kernel(token_ids, glove_table):
    B, S = token_ids.shape
    V, D = glove_table.shape
    n = B * S
    assert n % (NUM_WORKERS * CHUNK) == 0
    n_chunks = n // CHUNK
    chunks_per_w = n_chunks // NUM_WORKERS
    ids = token_ids.astype(jnp.int32).reshape(NUM_WORKERS, chunks_per_w, CHUNK)

    r1 = lax.slice(glove_table, (0, 0), (V, CH_SPLIT))

    run_t = pl.kernel(
        _transpose_body,
        out_type=jax.ShapeDtypeStruct((V, R2_W), jnp.float32),
        mesh=plsc.VectorSubcoreMesh(core_axis_name="c", subcore_axis_name="s"),
        scratch_types=[
            pltpu.VMEM((D, BLK), jnp.float32),
            pltpu.VMEM((D, BLK), jnp.float32),
            pltpu.VMEM((BLK, R2_W), jnp.float32),
            pltpu.SemaphoreType.DMA,
        ],
        compiler_params=pltpu.CompilerParams(
            needs_layout_passes=False, disable_bounds_checks=True),
    )
    r2 = run_t(glove_table.T)

    run_g = pl.kernel(
        _gather_body,
        out_type=jax.ShapeDtypeStruct((n, ROW_W), jnp.float32),
        mesh=plsc.VectorSubcoreMesh(core_axis_name="c", subcore_axis_name="s"),
        scratch_types=[
            pltpu.VMEM((chunks_per_w, CHUNK), jnp.int32),
            pltpu.VMEM((CHUNK, ROW_W), jnp.float32),
            pltpu.VMEM((CHUNK, ROW_W), jnp.float32),
            pltpu.SemaphoreType.DMA,
        ],
    )
    wide = run_g(ids, r1, r2)
    return lax.slice(wide, (0, 0), (n, EMBED_DIM)).reshape(B, S, EMBED_DIM)
